# baseline (speedup 1.0000x reference)
import math

import jax
import jax.numpy as jnp
from jax.experimental import pallas as pl
from jax.experimental.pallas import tpu as pltpu

DIM = 1024
HEADS = 16
BLOCK = 128
DH = DIM // HEADS
SEQ = 2048
NMODS = 3

TB = 512
FTB = 512


def _attn_body(x, wqkv, wo, bqkv, bo):
    f32 = jnp.float32
    bf16 = jnp.bfloat16
    qkv = jnp.dot(x, wqkv, preferred_element_type=f32) + bqkv
    qb = qkv[:, :DIM].astype(bf16)
    kb = qkv[:, DIM:2 * DIM].astype(bf16)
    vb = qkv[:, 2 * DIM:].astype(bf16)
    nsb = x.shape[0] // BLOCK
    grp = 2 if nsb % 2 == 0 else 1
    row_blocks = []
    for g in range(0, nsb, grp):
        scores = []
        for s in range(g, g + grp):
            qs = qb[s * BLOCK:(s + 1) * BLOCK]
            ks = kb[s * BLOCK:(s + 1) * BLOCK]
            for h in range(HEADS):
                qh = qs[:, h * DH:(h + 1) * DH]
                kh = ks[:, h * DH:(h + 1) * DH]
                scores.append(jax.lax.dot_general(
                    kh, qh, (((1,), (1,)), ((), ())),
                    preferred_element_type=f32))
        sc = jnp.concatenate(scores, axis=1)
        m = jnp.max(sc, axis=0, keepdims=True)
        e = jnp.exp(sc - m)
        p = e * (1.0 / jnp.sum(e, axis=0, keepdims=True))
        pb = p.astype(bf16)
        for s in range(g, g + grp):
            vs = vb[s * BLOCK:(s + 1) * BLOCK]
            heads = []
            for h in range(HEADS):
                t = (s - g) * HEADS + h
                ph = pb[:, t * BLOCK:(t + 1) * BLOCK]
                vh = vs[:, h * DH:(h + 1) * DH]
                heads.append(jax.lax.dot_general(
                    ph, vh, (((0,), (0,)), ((), ())),
                    preferred_element_type=f32))
            row_blocks.append(jnp.concatenate(heads, axis=-1))
    att = jnp.concatenate(row_blocks, axis=0)
    o = jnp.dot(att.astype(bf16), wo, preferred_element_type=f32) + bo
    return o.astype(bf16)


def _block_attn_kernel(x_ref, wqkv_ref, wo_ref, bqkv_ref, bo_ref, o_ref):
    o_ref[0] = _attn_body(x_ref[0], wqkv_ref[0], wo_ref[0],
                          bqkv_ref[0], bo_ref[0])


def _cross_fusion_kernel(ax_ref, wqkv_ref, wo_ref, bqkv_ref, bo_ref,
                         a_ref, wf_ref, bfus_ref, o_ref):
    f32 = jnp.float32
    cblk = _attn_body(ax_ref[0], wqkv_ref[0], wo_ref[0],
                      bqkv_ref[0], bo_ref[0])
    c = pl.program_id(1)

    @pl.when(c == 0)
    def _init():
        o_ref[...] = jnp.broadcast_to(bfus_ref[...], o_ref.shape)

    o_ref[...] += (
        jnp.dot(a_ref[0], wf_ref[c], preferred_element_type=f32)
        + jnp.dot(cblk, wf_ref[NMODS + c], preferred_element_type=f32))


def _block_attn(x, wqkv, wo, bqkv, bo):
    m, s, _ = x.shape
    ntb = s // TB
    return pl.pallas_call(
        _block_attn_kernel,
        grid=(m, ntb),
        in_specs=[
            pl.BlockSpec((1, TB, DIM), lambda i, j: (i, j, 0)),
            pl.BlockSpec((1, DIM, 3 * DIM), lambda i, j: (i, 0, 0)),
            pl.BlockSpec((1, DIM, DIM), lambda i, j: (i, 0, 0)),
            pl.BlockSpec((1, 1, 3 * DIM), lambda i, j: (i, 0, 0)),
            pl.BlockSpec((1, 1, DIM), lambda i, j: (i, 0, 0)),
        ],
        out_specs=pl.BlockSpec((1, TB, DIM), lambda i, j: (i, j, 0)),
        out_shape=jax.ShapeDtypeStruct((m, s, DIM), jnp.bfloat16),
    )(x, wqkv, wo, bqkv, bo)


def _cross_fusion(attended, cross_ops, wf, bfus):
    wqkv, wo, bqkv, bo = cross_ops
    across = attended.reshape(1, NMODS * SEQ, DIM)
    nt = SEQ // FTB
    return pl.pallas_call(
        _cross_fusion_kernel,
        grid=(nt, NMODS),
        in_specs=[
            pl.BlockSpec((1, FTB, DIM), lambda i, c: (0, c * nt + i, 0)),
            pl.BlockSpec((1, DIM, 3 * DIM), lambda i, c: (0, 0, 0)),
            pl.BlockSpec((1, DIM, DIM), lambda i, c: (0, 0, 0)),
            pl.BlockSpec((1, 1, 3 * DIM), lambda i, c: (0, 0, 0)),
            pl.BlockSpec((1, 1, DIM), lambda i, c: (0, 0, 0)),
            pl.BlockSpec((1, FTB, DIM), lambda i, c: (c, i, 0)),
            pl.BlockSpec((2 * NMODS, DIM, DIM), lambda i, c: (0, 0, 0)),
            pl.BlockSpec((1, DIM), lambda i, c: (0, 0)),
        ],
        out_specs=pl.BlockSpec((FTB, DIM), lambda i, c: (i, 0)),
        out_shape=jax.ShapeDtypeStruct((SEQ, DIM), jnp.float32),
    )(across, wqkv, wo, bqkv, bo, attended, wf, bfus)


def _attn_operands(plist):
    scale = 1.0 / math.sqrt(DH)
    wqkv = jnp.stack([
        jnp.concatenate([p["Wq"] * scale, p["Wk"], p["Wv"]], axis=1)
        for p in plist]).astype(jnp.bfloat16)
    wo = jnp.stack([p["Wo"] for p in plist]).astype(jnp.bfloat16)
    bqkv = jnp.stack([
        jnp.concatenate([p["bq"] * scale, p["bk"], p["bv"]])
        for p in plist]).astype(jnp.float32).reshape(len(plist), 1, 3 * DIM)
    bo = jnp.stack([p["bo"] for p in plist]).astype(
        jnp.float32).reshape(len(plist), 1, DIM)
    return wqkv, wo, bqkv, bo


def kernel(text, visual, audio, params):
    bf16 = jnp.bfloat16
    x = jnp.stack([text[0], visual[0], audio[0]]).astype(bf16)
    mod_ops = _attn_operands([params[m + "_attn"]
                              for m in ("text", "visual", "audio")])
    cross_ops = _attn_operands([params["cross_attn"]])
    fw = params["fusion_weights"].astype(jnp.float32)
    scales = jnp.concatenate([fw, fw]).reshape(2 * NMODS, 1, 1)
    wf = (params["fusion_W"].reshape(2 * NMODS, DIM, DIM) * scales).astype(bf16)
    bfus = params["fusion_b"].astype(jnp.float32).reshape(1, DIM)

    attended = _block_attn(x, *mod_ops)
    out = _cross_fusion(attended, cross_ops, wf, bfus)
    return out.reshape(1, SEQ, DIM)


# speedup vs baseline: 1.0092x; 1.0092x over previous
import math

import jax
import jax.numpy as jnp
from jax.experimental import pallas as pl
from jax.experimental.pallas import tpu as pltpu

DIM = 1024
HEADS = 16
BLOCK = 128
DH = DIM // HEADS
SEQ = 2048
NMODS = 3

TB = 512
FTB = 512


def _attn_body(x, wqkv, wo, bqkv, bo):
    f32 = jnp.float32
    bf16 = jnp.bfloat16
    qkv = jnp.dot(x, wqkv, preferred_element_type=f32) + bqkv
    qb = qkv[:, :DIM].astype(bf16)
    kb = qkv[:, DIM:2 * DIM].astype(bf16)
    vb = qkv[:, 2 * DIM:].astype(bf16)
    nsb = x.shape[0] // BLOCK
    grp = nsb
    row_blocks = []
    for g in range(0, nsb, grp):
        scores = []
        for s in range(g, g + grp):
            qs = qb[s * BLOCK:(s + 1) * BLOCK]
            ks = kb[s * BLOCK:(s + 1) * BLOCK]
            for h in range(HEADS):
                qh = qs[:, h * DH:(h + 1) * DH]
                kh = ks[:, h * DH:(h + 1) * DH]
                scores.append(jax.lax.dot_general(
                    kh, qh, (((1,), (1,)), ((), ())),
                    preferred_element_type=f32))
        sc = jnp.concatenate(scores, axis=1)
        m = jnp.max(sc, axis=0, keepdims=True)
        e = jnp.exp(sc - m)
        p = e * (1.0 / jnp.sum(e, axis=0, keepdims=True))
        pb = p.astype(bf16)
        for s in range(g, g + grp):
            vs = vb[s * BLOCK:(s + 1) * BLOCK]
            heads = []
            for h in range(HEADS):
                t = (s - g) * HEADS + h
                ph = pb[:, t * BLOCK:(t + 1) * BLOCK]
                vh = vs[:, h * DH:(h + 1) * DH]
                heads.append(jax.lax.dot_general(
                    ph, vh, (((0,), (0,)), ((), ())),
                    preferred_element_type=f32))
            row_blocks.append(jnp.concatenate(heads, axis=-1))
    att = jnp.concatenate(row_blocks, axis=0)
    o = jnp.dot(att.astype(bf16), wo, preferred_element_type=f32) + bo
    return o.astype(bf16)


def _block_attn_kernel(x_ref, wqkv_ref, wo_ref, bqkv_ref, bo_ref, o_ref):
    o_ref[0] = _attn_body(x_ref[0], wqkv_ref[0], wo_ref[0],
                          bqkv_ref[0], bo_ref[0])


def _cross_fusion_kernel(ax_ref, wqkv_ref, wo_ref, bqkv_ref, bo_ref,
                         a_ref, wf_ref, bfus_ref, o_ref):
    f32 = jnp.float32
    cblk = _attn_body(ax_ref[0], wqkv_ref[0], wo_ref[0],
                      bqkv_ref[0], bo_ref[0])
    c = pl.program_id(1)

    @pl.when(c == 0)
    def _init():
        o_ref[...] = jnp.broadcast_to(bfus_ref[...], o_ref.shape)

    o_ref[...] += (
        jnp.dot(a_ref[0], wf_ref[c], preferred_element_type=f32)
        + jnp.dot(cblk, wf_ref[NMODS + c], preferred_element_type=f32))


def _block_attn(x, wqkv, wo, bqkv, bo):
    m, s, _ = x.shape
    ntb = s // TB
    return pl.pallas_call(
        _block_attn_kernel,
        grid=(m, ntb),
        in_specs=[
            pl.BlockSpec((1, TB, DIM), lambda i, j: (i, j, 0)),
            pl.BlockSpec((1, DIM, 3 * DIM), lambda i, j: (i, 0, 0)),
            pl.BlockSpec((1, DIM, DIM), lambda i, j: (i, 0, 0)),
            pl.BlockSpec((1, 1, 3 * DIM), lambda i, j: (i, 0, 0)),
            pl.BlockSpec((1, 1, DIM), lambda i, j: (i, 0, 0)),
        ],
        out_specs=pl.BlockSpec((1, TB, DIM), lambda i, j: (i, j, 0)),
        out_shape=jax.ShapeDtypeStruct((m, s, DIM), jnp.bfloat16),
    )(x, wqkv, wo, bqkv, bo)


def _cross_fusion(attended, cross_ops, wf, bfus):
    wqkv, wo, bqkv, bo = cross_ops
    across = attended.reshape(1, NMODS * SEQ, DIM)
    nt = SEQ // FTB
    return pl.pallas_call(
        _cross_fusion_kernel,
        grid=(nt, NMODS),
        in_specs=[
            pl.BlockSpec((1, FTB, DIM), lambda i, c: (0, c * nt + i, 0)),
            pl.BlockSpec((1, DIM, 3 * DIM), lambda i, c: (0, 0, 0)),
            pl.BlockSpec((1, DIM, DIM), lambda i, c: (0, 0, 0)),
            pl.BlockSpec((1, 1, 3 * DIM), lambda i, c: (0, 0, 0)),
            pl.BlockSpec((1, 1, DIM), lambda i, c: (0, 0, 0)),
            pl.BlockSpec((1, FTB, DIM), lambda i, c: (c, i, 0)),
            pl.BlockSpec((2 * NMODS, DIM, DIM), lambda i, c: (0, 0, 0)),
            pl.BlockSpec((1, DIM), lambda i, c: (0, 0)),
        ],
        out_specs=pl.BlockSpec((FTB, DIM), lambda i, c: (i, 0)),
        out_shape=jax.ShapeDtypeStruct((SEQ, DIM), jnp.float32),
    )(across, wqkv, wo, bqkv, bo, attended, wf, bfus)


def _attn_operands(plist):
    scale = 1.0 / math.sqrt(DH)
    wqkv = jnp.stack([
        jnp.concatenate([p["Wq"] * scale, p["Wk"], p["Wv"]], axis=1)
        for p in plist]).astype(jnp.bfloat16)
    wo = jnp.stack([p["Wo"] for p in plist]).astype(jnp.bfloat16)
    bqkv = jnp.stack([
        jnp.concatenate([p["bq"] * scale, p["bk"], p["bv"]])
        for p in plist]).astype(jnp.float32).reshape(len(plist), 1, 3 * DIM)
    bo = jnp.stack([p["bo"] for p in plist]).astype(
        jnp.float32).reshape(len(plist), 1, DIM)
    return wqkv, wo, bqkv, bo


def kernel(text, visual, audio, params):
    bf16 = jnp.bfloat16
    x = jnp.stack([text[0], visual[0], audio[0]]).astype(bf16)
    mod_ops = _attn_operands([params[m + "_attn"]
                              for m in ("text", "visual", "audio")])
    cross_ops = _attn_operands([params["cross_attn"]])
    fw = params["fusion_weights"].astype(jnp.float32)
    scales = jnp.concatenate([fw, fw]).reshape(2 * NMODS, 1, 1)
    wf = (params["fusion_W"].reshape(2 * NMODS, DIM, DIM) * scales).astype(bf16)
    bfus = params["fusion_b"].astype(jnp.float32).reshape(1, DIM)

    attended = _block_attn(x, *mod_ops)
    out = _cross_fusion(attended, cross_ops, wf, bfus)
    return out.reshape(1, SEQ, DIM)
